# baseline (speedup 1.0000x reference)
import functools

import jax
import jax.numpy as jnp
import numpy as np
from jax import lax
from jax.experimental import pallas as pl
from jax.experimental.pallas import tpu as pltpu
from jax.experimental.pallas import tpu_sc as plsc

N_HEADS = 12
NUM_BUCKETS = 32
MAX_DISTANCE = 128
MAX_EXACT = NUM_BUCKETS // 2
NEG = float(np.finfo(np.float32).min)
BUCKET_SAT = 113

BQ = 256
BK = 512


def _embed_gather(table, idx):
    V, D = table.shape
    B = idx.shape[0]
    info = plsc.get_sparse_core_info()
    nw = info.num_cores * info.num_subcores
    assert B % (8 * nw) == 0 and D % info.num_lanes == 0
    b_per_w = B // nw
    mesh = plsc.VectorSubcoreMesh(core_axis_name="c", subcore_axis_name="s")

    @functools.partial(
        pl.kernel,
        mesh=mesh,
        out_type=jax.ShapeDtypeStruct((B, D), jnp.float32),
        scratch_types=[
            pltpu.VMEM((b_per_w,), jnp.int32),
            pltpu.VMEM((b_per_w, D), jnp.float32),
            pltpu.SemaphoreType.DMA,
        ],
    )
    def gather_kernel(table_hbm, idx_hbm, out_hbm, idx_v, rows_v, sem):
        wid = lax.axis_index("s") * info.num_cores + lax.axis_index("c")
        base = wid * b_per_w
        pltpu.sync_copy(idx_hbm.at[pl.ds(base, b_per_w)], idx_v)
        pltpu.async_copy(table_hbm.at[idx_v], rows_v, sem).wait()
        pltpu.sync_copy(rows_v, out_hbm.at[pl.ds(base, b_per_w)])

    return gather_kernel(table, idx)


def _bias_mask_kernel(dmask_ref, emask_ref, rtt_ref, epb_ref, ehs_ref,
                      bias_ref, dec_ref, enc_ref, epb_out_ref, ehs_out_ref,
                      sem_epb, sem_ehs):
    q = pl.program_id(0)
    k = pl.program_id(1)

    @pl.when((q == 0) & (k == 0))
    def _start_copies():
        pltpu.make_async_copy(epb_ref, epb_out_ref, sem_epb).start()
        pltpu.make_async_copy(ehs_ref, ehs_out_ref, sem_ehs).start()

    @pl.when((q == pl.num_programs(0) - 1) & (k == pl.num_programs(1) - 1))
    def _wait_copies():
        pltpu.make_async_copy(epb_ref, epb_out_ref, sem_epb).wait()
        pltpu.make_async_copy(ehs_ref, ehs_out_ref, sem_ehs).wait()
    q0 = q * BQ
    k0 = k * BK
    qi = q0 + lax.broadcasted_iota(jnp.int32, (BQ, BK), 0)
    ki = k0 + lax.broadcasted_iota(jnp.int32, (BQ, BK), 1)

    m = dmask_ref[0, :]
    causal = (ki <= qi).astype(jnp.float32)
    dec_tile = (1.0 - causal * m[None, :]) * NEG
    dec_ref[...] = dec_tile
    enc_ref[...] = (1.0 - emask_ref[...]) * NEG

    dmax = q0 + (BQ - 1) - k0
    dmin = q0 - (k0 + (BK - 1))
    is_band = (dmax > 0) & (dmin < BUCKET_SAT)

    @pl.when(is_band)
    def _band():
        d0 = q0 - k0
        kk_row = lax.broadcasted_iota(jnp.int32, (1, BK), 1)

        def bucket_row(d):
            rp = jnp.maximum(d, 0)
            rpf = jnp.maximum(rp, 1).astype(jnp.float32)
            rel_large = MAX_EXACT + (
                jnp.log(rpf / MAX_EXACT)
                / np.log(MAX_DISTANCE / MAX_EXACT)
                * (NUM_BUCKETS - MAX_EXACT)
            ).astype(jnp.int32)
            rel_large = jnp.minimum(rel_large, NUM_BUCKETS - 1)
            return jnp.where(rp < MAX_EXACT, rp, rel_large)

        broll = bucket_row(jnp.where(d0 > 0, d0, d0 + BK) - kk_row)

        qq = lax.broadcasted_iota(jnp.int32, (BQ, BK), 0)
        kk = lax.broadcasted_iota(jnp.int32, (BQ, BK), 1)
        unwrapped = kk >= qq
        const_on_unwrapped = d0 <= 0
        for hh in range(N_HEADS):
            v = jnp.zeros((1, BK), jnp.float32) + rtt_ref[hh, NUM_BUCKETS - 1]
            for b in range(NUM_BUCKETS - 1):
                v = jnp.where(broll == b, rtt_ref[hh, b], v)
            r = pltpu.roll(jnp.broadcast_to(v, (BQ, BK)), 0, 1,
                           stride=1, stride_axis=0)
            cflat = jnp.where(const_on_unwrapped, rtt_ref[hh, 0],
                              rtt_ref[hh, NUM_BUCKETS - 1])
            use_roll = unwrapped != const_on_unwrapped
            bias_ref[hh] = jnp.where(use_roll, r, cflat) + dec_tile

    @pl.when(jnp.logical_not(is_band))
    def _flat():
        for hh in range(N_HEADS):
            c = jnp.where(dmin >= BUCKET_SAT, rtt_ref[hh, NUM_BUCKETS - 1],
                          rtt_ref[hh, 0])
            bias_ref[hh] = dec_tile + c


def _bias_masks(dmask, emask, rtt, epb, ehs, s):
    import math
    assert math.gcd(BQ, BK) >= BUCKET_SAT and BQ <= BK
    grid = (s // BQ, s // BK)
    return pl.pallas_call(
        _bias_mask_kernel,
        grid=grid,
        in_specs=[
            pl.BlockSpec((1, BK), lambda q, k: (0, k)),
            pl.BlockSpec((1, BK), lambda q, k: (0, k)),
            pl.BlockSpec((N_HEADS, NUM_BUCKETS), lambda q, k: (0, 0)),
            pl.BlockSpec(memory_space=pl.ANY),
            pl.BlockSpec(memory_space=pl.ANY),
        ],
        out_specs=[
            pl.BlockSpec((N_HEADS, BQ, BK), lambda q, k: (0, q, k)),
            pl.BlockSpec((BQ, BK), lambda q, k: (q, k)),
            pl.BlockSpec((1, BK), lambda q, k: (0, k)),
            pl.BlockSpec(memory_space=pl.ANY),
            pl.BlockSpec(memory_space=pl.ANY),
        ],
        out_shape=[
            jax.ShapeDtypeStruct((N_HEADS, s, s), jnp.float32),
            jax.ShapeDtypeStruct((s, s), jnp.float32),
            jax.ShapeDtypeStruct((1, s), jnp.float32),
            jax.ShapeDtypeStruct(epb.shape, jnp.float32),
            jax.ShapeDtypeStruct(ehs.shape, jnp.float32),
        ],
        scratch_shapes=[pltpu.SemaphoreType.DMA, pltpu.SemaphoreType.DMA],
    )(dmask, emask, rtt, epb, ehs)


def kernel(encoder_hidden_states, encoder_position_bias, decoder_input_ids,
           decoder_attention_mask, encoder_attention_mask, embedding,
           relative_attention_bias):
    batch, s = decoder_input_ids.shape
    assert batch == 1 and encoder_attention_mask.shape[1] == s

    ids = decoder_input_ids.reshape(-1).astype(jnp.int32)
    dhs = _embed_gather(embedding, ids).reshape(batch, s, embedding.shape[1])

    rtt = relative_attention_bias.T
    bias, dec_ext, enc_ext, epb_out, ehs_out = _bias_masks(
        decoder_attention_mask, encoder_attention_mask, rtt,
        encoder_position_bias, encoder_hidden_states, s)

    return (
        ehs_out,
        epb_out,
        dhs,
        enc_ext.reshape(1, 1, 1, s),
        dec_ext.reshape(1, 1, s, s),
        bias.reshape(1, N_HEADS, s, s),
    )


# speedup vs baseline: 29.0303x; 29.0303x over previous
import functools

import jax
import jax.numpy as jnp
import numpy as np
from jax import lax
from jax.experimental import pallas as pl
from jax.experimental.pallas import tpu as pltpu
from jax.experimental.pallas import tpu_sc as plsc

N_HEADS = 12
NUM_BUCKETS = 32
MAX_DISTANCE = 128
MAX_EXACT = NUM_BUCKETS // 2
NEG = float(np.finfo(np.float32).min)
BUCKET_SAT = 113

BQ = 256
BK = 512


def _embed_gather(table, idx):
    V, D = table.shape
    B = idx.shape[0]
    info = plsc.get_sparse_core_info()
    nw = info.num_cores * info.num_subcores
    assert B % (8 * nw) == 0 and D % info.num_lanes == 0
    b_per_w = B // nw
    mesh = plsc.VectorSubcoreMesh(core_axis_name="c", subcore_axis_name="s")

    @functools.partial(
        pl.kernel,
        mesh=mesh,
        out_type=jax.ShapeDtypeStruct((B, D), jnp.float32),
        scratch_types=[
            pltpu.VMEM((b_per_w,), jnp.int32),
            pltpu.VMEM((b_per_w, D), jnp.float32),
            pltpu.SemaphoreType.DMA,
        ],
    )
    def gather_kernel(table_hbm, idx_hbm, out_hbm, idx_v, rows_v, sem):
        wid = lax.axis_index("s") * info.num_cores + lax.axis_index("c")
        base = wid * b_per_w
        pltpu.sync_copy(idx_hbm.at[pl.ds(base, b_per_w)], idx_v)
        pltpu.async_copy(table_hbm.at[idx_v], rows_v, sem).wait()
        pltpu.sync_copy(rows_v, out_hbm.at[pl.ds(base, b_per_w)])

    return gather_kernel(table, idx)


def _bias_mask_kernel(dmask_ref, emask_ref, rtt_ref, epb_ref, ehs_ref,
                      bias_ref, dec_ref, enc_ref, epb_out_ref, ehs_out_ref):
    epb_out_ref[...] = epb_ref[...]
    ehs_out_ref[...] = ehs_ref[...]
    q = pl.program_id(0)
    k = pl.program_id(1)
    q0 = q * BQ
    k0 = k * BK
    qi = q0 + lax.broadcasted_iota(jnp.int32, (BQ, BK), 0)
    ki = k0 + lax.broadcasted_iota(jnp.int32, (BQ, BK), 1)

    m = dmask_ref[0, :]
    causal = (ki <= qi).astype(jnp.float32)
    dec_tile = (1.0 - causal * m[None, :]) * NEG
    dec_ref[...] = dec_tile
    enc_ref[...] = (1.0 - emask_ref[...]) * NEG

    dmax = q0 + (BQ - 1) - k0
    dmin = q0 - (k0 + (BK - 1))
    is_band = (dmax > 0) & (dmin < BUCKET_SAT)

    @pl.when(is_band)
    def _band():
        d0 = q0 - k0
        kk_row = lax.broadcasted_iota(jnp.int32, (1, BK), 1)

        def bucket_row(d):
            rp = jnp.maximum(d, 0)
            rpf = jnp.maximum(rp, 1).astype(jnp.float32)
            rel_large = MAX_EXACT + (
                jnp.log(rpf / MAX_EXACT)
                / np.log(MAX_DISTANCE / MAX_EXACT)
                * (NUM_BUCKETS - MAX_EXACT)
            ).astype(jnp.int32)
            rel_large = jnp.minimum(rel_large, NUM_BUCKETS - 1)
            return jnp.where(rp < MAX_EXACT, rp, rel_large)

        broll = bucket_row(jnp.where(d0 > 0, d0, d0 + BK) - kk_row)

        qq = lax.broadcasted_iota(jnp.int32, (BQ, BK), 0)
        kk = lax.broadcasted_iota(jnp.int32, (BQ, BK), 1)
        unwrapped = kk >= qq
        const_on_unwrapped = d0 <= 0
        for hh in range(N_HEADS):
            v = jnp.zeros((1, BK), jnp.float32) + rtt_ref[hh, NUM_BUCKETS - 1]
            for b in range(NUM_BUCKETS - 1):
                v = jnp.where(broll == b, rtt_ref[hh, b], v)
            r = pltpu.roll(jnp.broadcast_to(v, (BQ, BK)), 0, 1,
                           stride=1, stride_axis=0)
            cflat = jnp.where(const_on_unwrapped, rtt_ref[hh, 0],
                              rtt_ref[hh, NUM_BUCKETS - 1])
            use_roll = unwrapped != const_on_unwrapped
            bias_ref[hh] = jnp.where(use_roll, r, cflat) + dec_tile

    @pl.when(jnp.logical_not(is_band))
    def _flat():
        for hh in range(N_HEADS):
            c = jnp.where(dmin >= BUCKET_SAT, rtt_ref[hh, NUM_BUCKETS - 1],
                          rtt_ref[hh, 0])
            bias_ref[hh] = dec_tile + c


def _bias_masks(dmask, emask, rtt, epb, ehs, s):
    import math
    assert math.gcd(BQ, BK) >= BUCKET_SAT and BQ <= BK
    ehs_dm = ehs.shape[2]
    grid = (s // BQ, s // BK)
    return pl.pallas_call(
        _bias_mask_kernel,
        grid=grid,
        in_specs=[
            pl.BlockSpec((1, BK), lambda q, k: (0, k)),
            pl.BlockSpec((1, BK), lambda q, k: (0, k)),
            pl.BlockSpec((N_HEADS, NUM_BUCKETS), lambda q, k: (0, 0)),
            pl.BlockSpec((1, N_HEADS, BQ, BK), lambda q, k: (0, 0, q, k)),
            pl.BlockSpec((1, BQ, ehs_dm), lambda q, k: (0, q, 0)),
        ],
        out_specs=[
            pl.BlockSpec((N_HEADS, BQ, BK), lambda q, k: (0, q, k)),
            pl.BlockSpec((BQ, BK), lambda q, k: (q, k)),
            pl.BlockSpec((1, BK), lambda q, k: (0, k)),
            pl.BlockSpec((1, N_HEADS, BQ, BK), lambda q, k: (0, 0, q, k)),
            pl.BlockSpec((1, BQ, ehs_dm), lambda q, k: (0, q, 0)),
        ],
        out_shape=[
            jax.ShapeDtypeStruct((N_HEADS, s, s), jnp.float32),
            jax.ShapeDtypeStruct((s, s), jnp.float32),
            jax.ShapeDtypeStruct((1, s), jnp.float32),
            jax.ShapeDtypeStruct(epb.shape, jnp.float32),
            jax.ShapeDtypeStruct(ehs.shape, jnp.float32),
        ],
    )(dmask, emask, rtt, epb, ehs)


def kernel(encoder_hidden_states, encoder_position_bias, decoder_input_ids,
           decoder_attention_mask, encoder_attention_mask, embedding,
           relative_attention_bias):
    batch, s = decoder_input_ids.shape
    assert batch == 1 and encoder_attention_mask.shape[1] == s

    ids = decoder_input_ids.reshape(-1).astype(jnp.int32)
    dhs = _embed_gather(embedding, ids).reshape(batch, s, embedding.shape[1])

    rtt = relative_attention_bias.T
    bias, dec_ext, enc_ext, epb_out, ehs_out = _bias_masks(
        decoder_attention_mask, encoder_attention_mask, rtt,
        encoder_position_bias, encoder_hidden_states, s)

    return (
        ehs_out,
        epb_out,
        dhs,
        enc_ext.reshape(1, 1, 1, s),
        dec_ext.reshape(1, 1, s, s),
        bias.reshape(1, N_HEADS, s, s),
    )


# speedup vs baseline: 29.0743x; 1.0015x over previous
import functools
import math

import jax
import jax.numpy as jnp
import numpy as np
from jax import lax
from jax.experimental import pallas as pl
from jax.experimental.pallas import tpu as pltpu
from jax.experimental.pallas import tpu_sc as plsc

N_HEADS = 12
NUM_BUCKETS = 32
MAX_DISTANCE = 128
MAX_EXACT = NUM_BUCKETS // 2
NEG = float(np.finfo(np.float32).min)
BUCKET_SAT = 113

BQ = 256
BK = 512


def _embed_gather(table, idx):
    V, D = table.shape
    B = idx.shape[0]
    info = plsc.get_sparse_core_info()
    nw = info.num_cores * info.num_subcores
    assert B % (8 * nw) == 0 and D % info.num_lanes == 0
    b_per_w = B // nw
    mesh = plsc.VectorSubcoreMesh(core_axis_name="c", subcore_axis_name="s")

    @functools.partial(
        pl.kernel,
        mesh=mesh,
        out_type=jax.ShapeDtypeStruct((B, D), jnp.float32),
        scratch_types=[
            pltpu.VMEM((b_per_w,), jnp.int32),
            pltpu.VMEM((b_per_w, D), jnp.float32),
            pltpu.SemaphoreType.DMA,
        ],
    )
    def gather_kernel(table_hbm, idx_hbm, out_hbm, idx_v, rows_v, sem):
        wid = lax.axis_index("s") * info.num_cores + lax.axis_index("c")
        base = wid * b_per_w
        pltpu.sync_copy(idx_hbm.at[pl.ds(base, b_per_w)], idx_v)
        pltpu.async_copy(table_hbm.at[idx_v], rows_v, sem).wait()
        pltpu.sync_copy(rows_v, out_hbm.at[pl.ds(base, b_per_w)])

    return gather_kernel(table, idx)


def _bias_mask_kernel(dmask_ref, emask_ref, rtt_ref, epb_ref, ehs_ref,
                      bias_ref, dec_ref, enc_ref, epb_out_ref, ehs_out_ref):
    epb_out_ref[...] = epb_ref[...]
    ehs_out_ref[...] = ehs_ref[...]
    q = pl.program_id(0)
    k = pl.program_id(1)
    q0 = q * BQ
    k0 = k * BK
    qi = q0 + lax.broadcasted_iota(jnp.int32, (BQ, BK), 0)
    ki = k0 + lax.broadcasted_iota(jnp.int32, (BQ, BK), 1)

    m = dmask_ref[0, :]
    causal = (ki <= qi).astype(jnp.float32)
    dec_tile = (1.0 - causal * m[None, :]) * NEG
    dec_ref[...] = dec_tile
    enc_ref[...] = (1.0 - emask_ref[...]) * NEG

    dmax = q0 + (BQ - 1) - k0
    dmin = q0 - (k0 + (BK - 1))
    is_band = (dmax > 0) & (dmin < BUCKET_SAT)

    @pl.when(is_band)
    def _band():
        d0 = q0 - k0
        kk_row = lax.broadcasted_iota(jnp.int32, (1, BK), 1)

        def bucket_row(d):
            rp = jnp.maximum(d, 0)
            rpf = jnp.maximum(rp, 1).astype(jnp.float32)
            rel_large = MAX_EXACT + (
                jnp.log(rpf / MAX_EXACT)
                / np.log(MAX_DISTANCE / MAX_EXACT)
                * (NUM_BUCKETS - MAX_EXACT)
            ).astype(jnp.int32)
            rel_large = jnp.minimum(rel_large, NUM_BUCKETS - 1)
            return jnp.where(rp < MAX_EXACT, rp, rel_large)

        broll = bucket_row(jnp.where(d0 > 0, d0, d0 + BK) - kk_row)

        qq = lax.broadcasted_iota(jnp.int32, (BQ, BK), 0)
        kk = lax.broadcasted_iota(jnp.int32, (BQ, BK), 1)
        unwrapped = kk >= qq
        const_on_unwrapped = d0 <= 0
        for hh in range(N_HEADS):
            v = jnp.zeros((1, BK), jnp.float32) + rtt_ref[hh, NUM_BUCKETS - 1]
            for b in range(NUM_BUCKETS - 1):
                v = jnp.where(broll == b, rtt_ref[hh, b], v)
            r = pltpu.roll(jnp.broadcast_to(v, (BQ, BK)), 0, 1,
                           stride=1, stride_axis=0)
            cflat = jnp.where(const_on_unwrapped, rtt_ref[hh, 0],
                              rtt_ref[hh, NUM_BUCKETS - 1])
            use_roll = unwrapped != const_on_unwrapped
            bias_ref[hh] = jnp.where(use_roll, r, cflat) + dec_tile

    @pl.when(jnp.logical_not(is_band))
    def _flat():
        for hh in range(N_HEADS):
            c = jnp.where(dmin >= BUCKET_SAT, rtt_ref[hh, NUM_BUCKETS - 1],
                          rtt_ref[hh, 0])
            bias_ref[hh] = dec_tile + c


def _bias_masks(dmask, emask, rtt, epb, ehs, s):
    assert math.gcd(BQ, BK) >= BUCKET_SAT and BQ <= BK
    ehs_dm = ehs.shape[2]
    grid = (s // BQ, s // BK)
    return pl.pallas_call(
        _bias_mask_kernel,
        grid=grid,
        in_specs=[
            pl.BlockSpec((1, BK), lambda q, k: (0, k)),
            pl.BlockSpec((1, BK), lambda q, k: (0, k)),
            pl.BlockSpec((N_HEADS, NUM_BUCKETS), lambda q, k: (0, 0)),
            pl.BlockSpec((1, N_HEADS, BQ, BK), lambda q, k: (0, 0, q, k)),
            pl.BlockSpec((1, BQ, ehs_dm), lambda q, k: (0, q, 0)),
        ],
        out_specs=[
            pl.BlockSpec((N_HEADS, BQ, BK), lambda q, k: (0, q, k)),
            pl.BlockSpec((BQ, BK), lambda q, k: (q, k)),
            pl.BlockSpec((1, BK), lambda q, k: (0, k)),
            pl.BlockSpec((1, N_HEADS, BQ, BK), lambda q, k: (0, 0, q, k)),
            pl.BlockSpec((1, BQ, ehs_dm), lambda q, k: (0, q, 0)),
        ],
        out_shape=[
            jax.ShapeDtypeStruct((N_HEADS, s, s), jnp.float32),
            jax.ShapeDtypeStruct((s, s), jnp.float32),
            jax.ShapeDtypeStruct((1, s), jnp.float32),
            jax.ShapeDtypeStruct(epb.shape, jnp.float32),
            jax.ShapeDtypeStruct(ehs.shape, jnp.float32),
        ],
    )(dmask, emask, rtt, epb, ehs)


def kernel(encoder_hidden_states, encoder_position_bias, decoder_input_ids,
           decoder_attention_mask, encoder_attention_mask, embedding,
           relative_attention_bias):
    batch, s = decoder_input_ids.shape
    assert batch == 1 and encoder_attention_mask.shape[1] == s

    ids = decoder_input_ids.reshape(-1).astype(jnp.int32)
    dhs = _embed_gather(embedding, ids).reshape(batch, s, embedding.shape[1])

    rtt = relative_attention_bias.T
    bias, dec_ext, enc_ext, epb_out, ehs_out = _bias_masks(
        decoder_attention_mask, encoder_attention_mask, rtt,
        encoder_position_bias, encoder_hidden_states, s)

    return (
        ehs_out,
        epb_out,
        dhs,
        enc_ext.reshape(1, 1, 1, s),
        dec_ext.reshape(1, 1, s, s),
        bias.reshape(1, N_HEADS, s, s),
    )
